# baseline (speedup 1.0000x reference)
import jax
import jax.numpy as jnp
from jax.experimental import pallas as pl
from jax.experimental.pallas import tpu as pltpu

B, K = 512, 256
BB = 128
HB = 16
N_IT = 50
EPSM = 1e-8
WSUM = 1275.0


def _cacis_kernel(s_ref, t_ref, c_ref, o_ref, mt_scr, u0_scr, em_scr):
    j = pl.program_id(1)
    base = j * HB

    c_all = c_ref[...]
    tot = jnp.sum(c_all, axis=(1, 2), keepdims=True)
    ii = jax.lax.broadcasted_iota(jnp.int32, (1, K, K), 1)
    jj = jax.lax.broadcasted_iota(jnp.int32, (1, K, K), 2)
    diag = jnp.sum(jnp.where(ii == jj, c_all, 0.0), axis=(1, 2), keepdims=True)
    eps3 = jnp.maximum((tot - diag) / float(K * K - K), EPSM)

    for b in range(HB):
        c_b = c_all[b]
        f_b = 0.5 * s_ref[pl.ds(base + b, 1), :]
        e_t = (f_b + c_b).T + f_b
        mmin = jnp.min(e_t, axis=(0, 1), keepdims=True)
        mt_b = jnp.exp((mmin - e_t) / eps3[b])
        mt_scr[pl.ds(base + b, 1)] = mt_b[None]
        u0_scr[pl.ds(base + b, 1)] = jnp.sum(mt_b, axis=0, keepdims=True)[None]
        em = jnp.concatenate([eps3[b], mmin], axis=1)
        em_scr[pl.ds(base + b, 1)] = em[None]

    @pl.when(j == 7)
    def _():
        lane2 = jax.lax.broadcasted_iota(jnp.int32, (BB, K), 1)

        def fw_step(u):
            idx1 = jnp.argmin(u, axis=1, keepdims=True)
            pieces = []
            for b in range(BB):
                sb = idx1[b, 0]
                pieces.append(mt_scr[b, pl.ds(sb, 1), :])
            cols = jnp.concatenate(pieces, axis=0)
            oh = (lane2 == idx1)
            return cols, oh

        u0 = u0_scr[:, 0, :]
        cols, oh = fw_step(u0)
        u = cols
        av = jnp.where(oh, 1.0, 0.0)

        def fw_body(t, carry):
            u, av = carry
            w = (t + 1).astype(jnp.float32)
            cols, oh = fw_step(u)
            return (u + w * cols, av + jnp.where(oh, w, 0.0))

        u, av = jax.lax.fori_loop(1, N_IT, fw_body, (u, av))

        em = em_scr[:, 0, :]
        eps8 = em[:, 0:1]
        mmin8 = em[:, 1:2]
        inv = 1.0 / (WSUM * WSUM)
        val8 = jnp.sum(av * u, axis=1, keepdims=True) * inv
        conj = -eps8 * jnp.log(val8) + mmin8

        t_all = t_ref[...]
        fy = jnp.sum(jnp.where(lane2 == t_all, s_ref[...], 0.0),
                     axis=1, keepdims=True)
        o_ref[...] = conj - fy


def _cacis_call(scores, t2, C, interpret=False):
    return pl.pallas_call(
        _cacis_kernel,
        grid=(B // BB, 8),
        in_specs=[
            pl.BlockSpec((BB, K), lambda i, j: (i, 0)),
            pl.BlockSpec((BB, 1), lambda i, j: (i, 0)),
            pl.BlockSpec((HB, K, K), lambda i, j: (8 * i + j, 0, 0)),
        ],
        out_specs=pl.BlockSpec((BB, 1), lambda i, j: (i, 0)),
        out_shape=jax.ShapeDtypeStruct((B, 1), jnp.float32),
        scratch_shapes=[
            pltpu.VMEM((BB, K, K), jnp.float32),
            pltpu.VMEM((BB, 1, K), jnp.float32),
            pltpu.VMEM((BB, 1, 2), jnp.float32),
        ],
        compiler_params=pltpu.CompilerParams(
            dimension_semantics=("arbitrary", "arbitrary"),
            vmem_limit_bytes=56 * 1024 * 1024,
        ),
        name="cacis_loss",
        interpret=interpret,
    )(scores, t2, C)


def kernel(scores, targets, C):
    t2 = targets.astype(jnp.int32).reshape(B, 1)
    per_batch = _cacis_call(scores, t2, C)
    return jnp.mean(per_batch)


# speedup vs baseline: 1.2727x; 1.2727x over previous
import jax
import jax.numpy as jnp
from jax.experimental import pallas as pl
from jax.experimental.pallas import tpu as pltpu

B, K = 512, 256
BB = 64
HB = 32
N_IT = 50
EPSM = 1e-8
WSUM = 1275.0


def _cacis_kernel(s_ref, t_ref, c_ref, o_ref, mt_scr, u0_scr, em_scr):
    j = pl.program_id(1)
    base = j * HB

    c_all = c_ref[...]
    tot = jnp.sum(c_all, axis=(1, 2), keepdims=True)
    ii = jax.lax.broadcasted_iota(jnp.int32, (1, K, K), 1)
    jj = jax.lax.broadcasted_iota(jnp.int32, (1, K, K), 2)
    diag = jnp.sum(jnp.where(ii == jj, c_all, 0.0), axis=(1, 2), keepdims=True)
    eps3 = jnp.maximum((tot - diag) / float(K * K - K), EPSM)

    for b in range(HB):
        c_b = c_all[b]
        f_b = 0.5 * s_ref[pl.ds(base + b, 1), :]
        e_t = (f_b + c_b).T + f_b
        mmin = jnp.min(e_t, axis=(0, 1), keepdims=True)
        mt_b = jnp.exp((mmin - e_t) / eps3[b])
        mt_scr[pl.ds(base + b, 1)] = mt_b[None]
        u0_scr[pl.ds(base + b, 1)] = jnp.sum(mt_b, axis=0, keepdims=True)[None]
        em = jnp.concatenate([eps3[b], mmin], axis=1)
        em_scr[pl.ds(base + b, 1)] = em[None]

    @pl.when(j == 1)
    def _():
        lane2 = jax.lax.broadcasted_iota(jnp.int32, (BB, K), 1)

        def fw_step(u):
            idx1 = jnp.argmin(u, axis=1, keepdims=True)
            pieces = []
            for b in range(BB):
                sb = idx1[b, 0]
                pieces.append(mt_scr[b, pl.ds(sb, 1), :])
            cols = jnp.concatenate(pieces, axis=0)
            oh = (lane2 == idx1)
            return cols, oh

        u0 = u0_scr[:, 0, :]
        cols, oh = fw_step(u0)
        u = cols
        av = jnp.where(oh, 1.0, 0.0)

        def fw_body(t, carry):
            u, av = carry
            w = (t + 1).astype(jnp.float32)
            cols, oh = fw_step(u)
            return (u + w * cols, av + jnp.where(oh, w, 0.0))

        u, av = jax.lax.fori_loop(1, N_IT, fw_body, (u, av), unroll=2)

        em = em_scr[:, 0, :]
        eps8 = em[:, 0:1]
        mmin8 = em[:, 1:2]
        inv = 1.0 / (WSUM * WSUM)
        val8 = jnp.sum(av * u, axis=1, keepdims=True) * inv
        conj = -eps8 * jnp.log(val8) + mmin8

        t_all = t_ref[...]
        fy = jnp.sum(jnp.where(lane2 == t_all, s_ref[...], 0.0),
                     axis=1, keepdims=True)
        o_ref[...] = conj - fy


def _cacis_call(scores, t2, C, interpret=False):
    return pl.pallas_call(
        _cacis_kernel,
        grid=(B // BB, 2),
        in_specs=[
            pl.BlockSpec((BB, K), lambda i, j: (i, 0)),
            pl.BlockSpec((BB, 1), lambda i, j: (i, 0)),
            pl.BlockSpec((HB, K, K), lambda i, j: (2 * i + j, 0, 0)),
        ],
        out_specs=pl.BlockSpec((BB, 1), lambda i, j: (i, 0)),
        out_shape=jax.ShapeDtypeStruct((B, 1), jnp.float32),
        scratch_shapes=[
            pltpu.VMEM((BB, K, K), jnp.float32),
            pltpu.VMEM((BB, 1, K), jnp.float32),
            pltpu.VMEM((BB, 1, 2), jnp.float32),
        ],
        compiler_params=pltpu.CompilerParams(
            dimension_semantics=("arbitrary", "arbitrary"),
            vmem_limit_bytes=56 * 1024 * 1024,
        ),
        name="cacis_loss",
        interpret=interpret,
    )(scores, t2, C)


def kernel(scores, targets, C):
    t2 = targets.astype(jnp.int32).reshape(B, 1)
    per_batch = _cacis_call(scores, t2, C)
    return jnp.mean(per_batch)


# speedup vs baseline: 1.2792x; 1.0051x over previous
import jax
import jax.numpy as jnp
from jax.experimental import pallas as pl
from jax.experimental.pallas import tpu as pltpu

B, K = 512, 256
BB = 64
HB = 32
N_IT = 50
EPSM = 1e-8
WSUM = 1275.0


def _cacis_kernel(s_ref, t_ref, c_ref, o_ref, mt_scr, u0_scr, em_scr):
    j = pl.program_id(1)
    base = j * HB

    c_all = c_ref[...]
    tot = jnp.sum(c_all, axis=(1, 2), keepdims=True)
    ii = jax.lax.broadcasted_iota(jnp.int32, (1, K, K), 1)
    jj = jax.lax.broadcasted_iota(jnp.int32, (1, K, K), 2)
    diag = jnp.sum(jnp.where(ii == jj, c_all, 0.0), axis=(1, 2), keepdims=True)
    eps3 = jnp.maximum((tot - diag) / float(K * K - K), EPSM)

    for b in range(HB):
        c_b = c_all[b]
        f_b = 0.5 * s_ref[pl.ds(base + b, 1), :]
        e_t = (f_b + c_b).T + f_b
        mmin = jnp.min(e_t, axis=(0, 1), keepdims=True)
        mt_b = jnp.exp((mmin - e_t) / eps3[b])
        mt_scr[pl.ds(base + b, 1)] = mt_b[None]
        u0_scr[pl.ds(base + b, 1)] = jnp.sum(mt_b, axis=0, keepdims=True)[None]
        em = jnp.concatenate([eps3[b], mmin], axis=1)
        em_scr[pl.ds(base + b, 1)] = em[None]

    @pl.when(j == 1)
    def _():
        lane2 = jax.lax.broadcasted_iota(jnp.int32, (BB, K), 1)

        def fw_step(u):
            idx1 = jnp.argmin(u, axis=1, keepdims=True)
            pieces = []
            for b in range(BB):
                sb = idx1[b, 0]
                pieces.append(mt_scr[b, pl.ds(sb, 1), :])
            cols = jnp.concatenate(pieces, axis=0)
            oh = (lane2 == idx1)
            return cols, oh

        u0 = u0_scr[:, 0, :]
        cols, oh = fw_step(u0)
        u = cols
        av = jnp.where(oh, 1.0, 0.0)

        def fw_body(t, carry):
            u, av = carry
            w = (t + 1).astype(jnp.float32)
            cols, oh = fw_step(u)
            return (u + w * cols, av + jnp.where(oh, w, 0.0))

        u, av = jax.lax.fori_loop(1, N_IT, fw_body, (u, av), unroll=4)

        em = em_scr[:, 0, :]
        eps8 = em[:, 0:1]
        mmin8 = em[:, 1:2]
        inv = 1.0 / (WSUM * WSUM)
        val8 = jnp.sum(av * u, axis=1, keepdims=True) * inv
        conj = -eps8 * jnp.log(val8) + mmin8

        t_all = t_ref[...]
        fy = jnp.sum(jnp.where(lane2 == t_all, s_ref[...], 0.0),
                     axis=1, keepdims=True)
        o_ref[...] = conj - fy


def _cacis_call(scores, t2, C, interpret=False):
    return pl.pallas_call(
        _cacis_kernel,
        grid=(B // BB, 2),
        in_specs=[
            pl.BlockSpec((BB, K), lambda i, j: (i, 0)),
            pl.BlockSpec((BB, 1), lambda i, j: (i, 0)),
            pl.BlockSpec((HB, K, K), lambda i, j: (2 * i + j, 0, 0)),
        ],
        out_specs=pl.BlockSpec((BB, 1), lambda i, j: (i, 0)),
        out_shape=jax.ShapeDtypeStruct((B, 1), jnp.float32),
        scratch_shapes=[
            pltpu.VMEM((BB, K, K), jnp.float32),
            pltpu.VMEM((BB, 1, K), jnp.float32),
            pltpu.VMEM((BB, 1, 2), jnp.float32),
        ],
        compiler_params=pltpu.CompilerParams(
            dimension_semantics=("arbitrary", "arbitrary"),
            vmem_limit_bytes=56 * 1024 * 1024,
        ),
        name="cacis_loss",
        interpret=interpret,
    )(scores, t2, C)


def kernel(scores, targets, C):
    t2 = targets.astype(jnp.int32).reshape(B, 1)
    per_batch = _cacis_call(scores, t2, C)
    return jnp.mean(per_batch)


# speedup vs baseline: 1.3289x; 1.0389x over previous
import jax
import jax.numpy as jnp
from jax.experimental import pallas as pl
from jax.experimental.pallas import tpu as pltpu

B, K = 512, 256
BB = 64
HB = 32
N_IT = 50
EPSM = 1e-8
WSUM = 1275.0


def _cacis_kernel(s_ref, t_ref, c_ref, o_ref, mt_scr, u0_scr, em_scr):
    j = pl.program_id(1)
    base = j * HB

    c_all = c_ref[...]
    tot = jnp.sum(c_all, axis=(1, 2), keepdims=True)
    ii = jax.lax.broadcasted_iota(jnp.int32, (1, K, K), 1)
    jj = jax.lax.broadcasted_iota(jnp.int32, (1, K, K), 2)
    diag = jnp.sum(jnp.where(ii == jj, c_all, 0.0), axis=(1, 2), keepdims=True)
    eps3 = jnp.maximum((tot - diag) / float(K * K - K), EPSM)

    for b in range(HB):
        c_b = c_all[b]
        f_b = 0.5 * s_ref[pl.ds(base + b, 1), :]
        e_t = (f_b + c_b).T + f_b
        mmin = jnp.min(e_t, axis=(0, 1), keepdims=True)
        mt_b = jnp.exp((mmin - e_t) / eps3[b])
        mt_scr[pl.ds(base + b, 1)] = mt_b[None]
        u0_scr[pl.ds(base + b, 1)] = jnp.sum(mt_b, axis=0, keepdims=True)[None]
        em = jnp.concatenate([eps3[b], mmin], axis=1)
        em_scr[pl.ds(base + b, 1)] = em[None]

    @pl.when(j == 1)
    def _():
        lane2 = jax.lax.broadcasted_iota(jnp.int32, (BB, K), 1)

        def fw_step(u):
            idx1 = jnp.argmin(u, axis=1, keepdims=True)
            pieces = []
            for b in range(BB):
                sb = idx1[b, 0]
                pieces.append(mt_scr[b, pl.ds(sb, 1), :])
            cols = jnp.concatenate(pieces, axis=0)
            oh = (lane2 == idx1)
            return cols, oh

        u0 = u0_scr[:, 0, :]
        cols, oh = fw_step(u0)
        u = cols
        av = jnp.where(oh, 1.0, 0.0)

        def fw_body(t, carry):
            u, av = carry
            w = (t + 1).astype(jnp.float32)
            cols, oh = fw_step(u)
            return (u + w * cols, av + jnp.where(oh, w, 0.0))

        u, av = jax.lax.fori_loop(1, N_IT, fw_body, (u, av), unroll=7)

        em = em_scr[:, 0, :]
        eps8 = em[:, 0:1]
        mmin8 = em[:, 1:2]
        inv = 1.0 / (WSUM * WSUM)
        val8 = jnp.sum(av * u, axis=1, keepdims=True) * inv
        conj = -eps8 * jnp.log(val8) + mmin8

        t_all = t_ref[...]
        fy = jnp.sum(jnp.where(lane2 == t_all, s_ref[...], 0.0),
                     axis=1, keepdims=True)
        o_ref[...] = conj - fy


def _cacis_call(scores, t2, C, interpret=False):
    return pl.pallas_call(
        _cacis_kernel,
        grid=(B // BB, 2),
        in_specs=[
            pl.BlockSpec((BB, K), lambda i, j: (i, 0)),
            pl.BlockSpec((BB, 1), lambda i, j: (i, 0)),
            pl.BlockSpec((HB, K, K), lambda i, j: (2 * i + j, 0, 0)),
        ],
        out_specs=pl.BlockSpec((BB, 1), lambda i, j: (i, 0)),
        out_shape=jax.ShapeDtypeStruct((B, 1), jnp.float32),
        scratch_shapes=[
            pltpu.VMEM((BB, K, K), jnp.float32),
            pltpu.VMEM((BB, 1, K), jnp.float32),
            pltpu.VMEM((BB, 1, 2), jnp.float32),
        ],
        compiler_params=pltpu.CompilerParams(
            dimension_semantics=("arbitrary", "arbitrary"),
            vmem_limit_bytes=56 * 1024 * 1024,
        ),
        name="cacis_loss",
        interpret=interpret,
    )(scores, t2, C)


def kernel(scores, targets, C):
    t2 = targets.astype(jnp.int32).reshape(B, 1)
    per_batch = _cacis_call(scores, t2, C)
    return jnp.mean(per_batch)


# speedup vs baseline: 1.4061x; 1.0581x over previous
import jax
import jax.numpy as jnp
from jax.experimental import pallas as pl
from jax.experimental.pallas import tpu as pltpu

B, K = 512, 256
BB = 64
HB = 32
N_IT = 50
EPSM = 1e-8
WSUM = 1275.0


def _cacis_kernel(s_ref, t_ref, c_ref, o_ref, mt_scr, u0_scr, em_scr):
    j = pl.program_id(1)
    base = j * HB

    c_all = c_ref[...]
    tot = jnp.sum(c_all, axis=(1, 2), keepdims=True)
    ii = jax.lax.broadcasted_iota(jnp.int32, (1, K, K), 1)
    jj = jax.lax.broadcasted_iota(jnp.int32, (1, K, K), 2)
    diag = jnp.sum(jnp.where(ii == jj, c_all, 0.0), axis=(1, 2), keepdims=True)
    eps3 = jnp.maximum((tot - diag) / float(K * K - K), EPSM)

    for b in range(HB):
        c_b = c_all[b]
        f_b = 0.5 * s_ref[pl.ds(base + b, 1), :]
        e_t = (f_b + c_b).T + f_b
        mmin = jnp.min(e_t, axis=(0, 1), keepdims=True)
        mt_b = jnp.exp((mmin - e_t) / eps3[b])
        mt_scr[pl.ds(base + b, 1)] = mt_b[None]
        u0_scr[pl.ds(base + b, 1)] = jnp.sum(mt_b, axis=0, keepdims=True)[None]
        em = jnp.concatenate([eps3[b], mmin], axis=1)
        em_scr[pl.ds(base + b, 1)] = em[None]

    @pl.when(j == 1)
    def _():
        lane2 = jax.lax.broadcasted_iota(jnp.int32, (BB, K), 1)

        def fw_step(u):
            idx1 = jnp.argmin(u, axis=1, keepdims=True)
            pieces = []
            for b in range(BB):
                sb = idx1[b, 0]
                pieces.append(mt_scr[b, pl.ds(sb, 1), :])
            cols = jnp.concatenate(pieces, axis=0)
            oh = (lane2 == idx1)
            return cols, oh

        u0 = u0_scr[:, 0, :]
        cols, oh = fw_step(u0)
        u = cols
        av = jnp.where(oh, 1.0, 0.0)

        def fw_body(t, carry):
            u, av = carry
            w = (t + 1).astype(jnp.float32)
            cols, oh = fw_step(u)
            return (u + w * cols, av + jnp.where(oh, w, 0.0))

        u, av = jax.lax.fori_loop(1, N_IT, fw_body, (u, av), unroll=49)

        em = em_scr[:, 0, :]
        eps8 = em[:, 0:1]
        mmin8 = em[:, 1:2]
        inv = 1.0 / (WSUM * WSUM)
        val8 = jnp.sum(av * u, axis=1, keepdims=True) * inv
        conj = -eps8 * jnp.log(val8) + mmin8

        t_all = t_ref[...]
        fy = jnp.sum(jnp.where(lane2 == t_all, s_ref[...], 0.0),
                     axis=1, keepdims=True)
        o_ref[...] = conj - fy


def _cacis_call(scores, t2, C, interpret=False):
    return pl.pallas_call(
        _cacis_kernel,
        grid=(B // BB, 2),
        in_specs=[
            pl.BlockSpec((BB, K), lambda i, j: (i, 0)),
            pl.BlockSpec((BB, 1), lambda i, j: (i, 0)),
            pl.BlockSpec((HB, K, K), lambda i, j: (2 * i + j, 0, 0)),
        ],
        out_specs=pl.BlockSpec((BB, 1), lambda i, j: (i, 0)),
        out_shape=jax.ShapeDtypeStruct((B, 1), jnp.float32),
        scratch_shapes=[
            pltpu.VMEM((BB, K, K), jnp.float32),
            pltpu.VMEM((BB, 1, K), jnp.float32),
            pltpu.VMEM((BB, 1, 2), jnp.float32),
        ],
        compiler_params=pltpu.CompilerParams(
            dimension_semantics=("arbitrary", "arbitrary"),
            vmem_limit_bytes=56 * 1024 * 1024,
        ),
        name="cacis_loss",
        interpret=interpret,
    )(scores, t2, C)


def kernel(scores, targets, C):
    t2 = targets.astype(jnp.int32).reshape(B, 1)
    per_batch = _cacis_call(scores, t2, C)
    return jnp.mean(per_batch)
